# Initial kernel scaffold; baseline (speedup 1.0000x reference)
import jax
import jax.numpy as jnp
from jax.experimental import pallas as pl


def kernel(features, adj, W0, a1_0, a2_0, Wc, a1_c, a2_c):
    raise NotImplementedError("write your pallas kernel here")


# speedup vs baseline: 1.9228x; 1.9228x over previous
import jax
import jax.numpy as jnp
from jax.experimental import pallas as pl

_NEG = -1e9


def _row_block(n: int, target: int) -> int:
    best = 8
    for d in range(8, min(n, target) + 1, 8):
        if n % d == 0:
            best = d
    return best if n % best == 0 else n


def _proj_kernel(x_ref, w0_ref, a1_ref, a2_ref, h0_ref, f1_ref, f2t_ref):
    x = x_ref[...]
    heads = w0_ref.shape[0]
    f1s, f2s = [], []
    for h in range(heads):
        hh = jnp.dot(x, w0_ref[h], preferred_element_type=jnp.float32)
        h0_ref[h] = hh
        f1s.append(jnp.sum(hh * a1_ref[h][None, :], axis=1, keepdims=True))
        f2s.append(jnp.sum(hh * a2_ref[h][None, :], axis=1, keepdims=True))
    f1_ref[...] = jnp.concatenate(f1s, axis=1)
    f2t_ref[...] = jnp.concatenate(f2s, axis=1).T


def _attn_block(adj_blk, f1_col, f2_row, h_all):
    z = f1_col + f2_row
    e = jnp.where(z >= 0, z, 0.2 * z)
    e = jnp.where(adj_blk > 0, e, _NEG)
    m = jnp.max(e, axis=1, keepdims=True)
    p = jnp.exp(e - m)
    s = jnp.sum(p, axis=1, keepdims=True)
    o = jnp.dot(p, h_all, preferred_element_type=jnp.float32)
    return o / s


def _layer1_kernel(adj_ref, h0_ref, f1_ref, f2t_ref, wc_ref, a1c_ref, a2c_ref,
                   hc_ref, f1c_ref, f2c_ref):
    adj = adj_ref[...]
    heads = h0_ref.shape[0]
    outs = []
    for h in range(heads):
        outs.append(_attn_block(adj, f1_ref[:, h:h + 1], f2t_ref[h:h + 1, :],
                                h0_ref[h]))
    x = jnp.concatenate(outs, axis=1)
    x = jnp.where(x > 0, x, jnp.exp(x) - 1.0)
    hc = jnp.dot(x, wc_ref[...], preferred_element_type=jnp.float32)
    hc_ref[...] = hc
    f1c_ref[...] = jnp.sum(hc * a1c_ref[...], axis=1, keepdims=True)
    f2c_ref[...] = jnp.sum(hc * a2c_ref[...], axis=1, keepdims=True)


def _cls_kernel(adj_ref, hc_ref, f1c_ref, f2c_ref, out_ref):
    f2c_row = f2c_ref[...].T
    out_ref[...] = _attn_block(adj_ref[...], f1c_ref[...], f2c_row,
                               hc_ref[...])


@jax.jit
def kernel(features, adj, W0, a1_0, a2_0, Wc, a1_c, a2_c):
    n, d_in = features.shape
    heads, _, f_out = W0.shape
    d_mid = heads * f_out
    n_cls = Wc.shape[1]

    h0, f1, f2t = pl.pallas_call(
        _proj_kernel,
        out_shape=[
            jax.ShapeDtypeStruct((heads, n, f_out), jnp.float32),
            jax.ShapeDtypeStruct((n, heads), jnp.float32),
            jax.ShapeDtypeStruct((heads, n), jnp.float32),
        ],
    )(features, W0, a1_0, a2_0)

    bi = _row_block(n, 80)
    hc, f1c, f2c = pl.pallas_call(
        _layer1_kernel,
        grid=(n // bi,),
        in_specs=[
            pl.BlockSpec((bi, n), lambda i: (i, 0)),
            pl.BlockSpec((heads, n, f_out), lambda i: (0, 0, 0)),
            pl.BlockSpec((bi, heads), lambda i: (i, 0)),
            pl.BlockSpec((heads, n), lambda i: (0, 0)),
            pl.BlockSpec((d_mid, n_cls), lambda i: (0, 0)),
            pl.BlockSpec((1, n_cls), lambda i: (0, 0)),
            pl.BlockSpec((1, n_cls), lambda i: (0, 0)),
        ],
        out_specs=[
            pl.BlockSpec((bi, n_cls), lambda i: (i, 0)),
            pl.BlockSpec((bi, 1), lambda i: (i, 0)),
            pl.BlockSpec((bi, 1), lambda i: (i, 0)),
        ],
        out_shape=[
            jax.ShapeDtypeStruct((n, n_cls), jnp.float32),
            jax.ShapeDtypeStruct((n, 1), jnp.float32),
            jax.ShapeDtypeStruct((n, 1), jnp.float32),
        ],
    )(adj, h0, f1, f2t, Wc, a1_c.reshape(1, n_cls), a2_c.reshape(1, n_cls))

    out = pl.pallas_call(
        _cls_kernel,
        grid=(n // bi,),
        in_specs=[
            pl.BlockSpec((bi, n), lambda i: (i, 0)),
            pl.BlockSpec((n, n_cls), lambda i: (0, 0)),
            pl.BlockSpec((bi, 1), lambda i: (i, 0)),
            pl.BlockSpec((n, 1), lambda i: (0, 0)),
        ],
        out_specs=pl.BlockSpec((bi, n_cls), lambda i: (i, 0)),
        out_shape=jax.ShapeDtypeStruct((n, n_cls), jnp.float32),
    )(adj, hc, f1c, f2c)
    return out


# speedup vs baseline: 2.2170x; 1.1530x over previous
import jax
import jax.numpy as jnp
from jax.experimental import pallas as pl

_NEG = -1e9


def _row_block(n: int, target: int) -> int:
    best = 8
    for d in range(8, min(n, target) + 1, 8):
        if n % d == 0:
            best = d
    return best if n % best == 0 else n


def _proj_kernel(x_ref, w0_ref, a1_ref, a2_ref, h0_ref, f1_ref, f2t_ref):
    x = x_ref[...]
    heads = w0_ref.shape[0]
    n = x.shape[0]
    ones = jnp.ones((n, 1), dtype=jnp.float32)
    f1s, f2s = [], []
    for h in range(heads):
        hh = jnp.dot(x, w0_ref[h], preferred_element_type=jnp.float32)
        h0_ref[h] = jnp.concatenate([hh, ones], axis=1)
        f1s.append(jnp.sum(hh * a1_ref[h][None, :], axis=1, keepdims=True))
        f2s.append(jnp.sum(hh * a2_ref[h][None, :], axis=1, keepdims=True))
    f1_ref[...] = jnp.concatenate(f1s, axis=1)
    f2t_ref[...] = jnp.concatenate(f2s, axis=1).T


def _attn_block(mask, f1_col, f2_row, h_aug):
    z = f1_col + f2_row
    e = jnp.maximum(z, 0.2 * z)
    e = jnp.where(mask, e, _NEG)
    m = jnp.max(e, axis=1, keepdims=True)
    p = jnp.exp(e - m)
    os = jnp.dot(p, h_aug, preferred_element_type=jnp.float32)
    f = h_aug.shape[1] - 1
    return os[:, :f] / os[:, f:]


def _layer1_kernel(adj_ref, h0_ref, f1_ref, f2t_ref, wc_ref, a1c_ref, a2c_ref,
                   hc_ref, f1c_ref, f2c_ref):
    mask = adj_ref[...] > 0
    heads = h0_ref.shape[0]
    outs = []
    for h in range(heads):
        outs.append(_attn_block(mask, f1_ref[:, h:h + 1], f2t_ref[h:h + 1, :],
                                h0_ref[h]))
    x = jnp.concatenate(outs, axis=1)
    x = jnp.where(x > 0, x, jnp.exp(x) - 1.0)
    hc = jnp.dot(x, wc_ref[...], preferred_element_type=jnp.float32)
    ones = jnp.ones((hc.shape[0], 1), dtype=jnp.float32)
    hc_ref[...] = jnp.concatenate([hc, ones], axis=1)
    f1c_ref[...] = jnp.sum(hc * a1c_ref[...], axis=1, keepdims=True)
    f2c_ref[...] = jnp.sum(hc * a2c_ref[...], axis=1, keepdims=True)


def _cls_kernel(adj_ref, hc_ref, f1c_ref, f2c_ref, out_ref):
    mask = adj_ref[...] > 0
    f2c_row = f2c_ref[...].T
    out_ref[...] = _attn_block(mask, f1c_ref[...], f2c_row, hc_ref[...])


@jax.jit
def kernel(features, adj, W0, a1_0, a2_0, Wc, a1_c, a2_c):
    n, d_in = features.shape
    heads, _, f_out = W0.shape
    d_mid = heads * f_out
    n_cls = Wc.shape[1]

    h0, f1, f2t = pl.pallas_call(
        _proj_kernel,
        out_shape=[
            jax.ShapeDtypeStruct((heads, n, f_out + 1), jnp.float32),
            jax.ShapeDtypeStruct((n, heads), jnp.float32),
            jax.ShapeDtypeStruct((heads, n), jnp.float32),
        ],
    )(features, W0, a1_0, a2_0)

    bi = _row_block(n, 80)
    hc, f1c, f2c = pl.pallas_call(
        _layer1_kernel,
        grid=(n // bi,),
        in_specs=[
            pl.BlockSpec((bi, n), lambda i: (i, 0)),
            pl.BlockSpec((heads, n, f_out + 1), lambda i: (0, 0, 0)),
            pl.BlockSpec((bi, heads), lambda i: (i, 0)),
            pl.BlockSpec((heads, n), lambda i: (0, 0)),
            pl.BlockSpec((d_mid, n_cls), lambda i: (0, 0)),
            pl.BlockSpec((1, n_cls), lambda i: (0, 0)),
            pl.BlockSpec((1, n_cls), lambda i: (0, 0)),
        ],
        out_specs=[
            pl.BlockSpec((bi, n_cls + 1), lambda i: (i, 0)),
            pl.BlockSpec((bi, 1), lambda i: (i, 0)),
            pl.BlockSpec((bi, 1), lambda i: (i, 0)),
        ],
        out_shape=[
            jax.ShapeDtypeStruct((n, n_cls + 1), jnp.float32),
            jax.ShapeDtypeStruct((n, 1), jnp.float32),
            jax.ShapeDtypeStruct((n, 1), jnp.float32),
        ],
    )(adj, h0, f1, f2t, Wc, a1_c.reshape(1, n_cls), a2_c.reshape(1, n_cls))

    bc = _row_block(n, 200)
    out = pl.pallas_call(
        _cls_kernel,
        grid=(n // bc,),
        in_specs=[
            pl.BlockSpec((bc, n), lambda i: (i, 0)),
            pl.BlockSpec((n, n_cls + 1), lambda i: (0, 0)),
            pl.BlockSpec((bc, 1), lambda i: (i, 0)),
            pl.BlockSpec((n, 1), lambda i: (0, 0)),
        ],
        out_specs=pl.BlockSpec((bc, n_cls), lambda i: (i, 0)),
        out_shape=jax.ShapeDtypeStruct((n, n_cls), jnp.float32),
    )(adj, hc, f1c, f2c)
    return out


# speedup vs baseline: 3.0037x; 1.3548x over previous
import jax
import jax.numpy as jnp
from jax.experimental import pallas as pl

_NEG = -1e9


def _row_block(n: int, target: int) -> int:
    best = 8
    for d in range(8, min(n, target) + 1, 8):
        if n % d == 0:
            best = d
    return best if n % best == 0 else n


def _proj_kernel(x_ref, w0_ref, a1_ref, a2_ref, h0_ref, f1_ref, f2t_ref):
    x = x_ref[...]
    heads = w0_ref.shape[0]
    n = x.shape[0]
    ones = jnp.ones((n, 1), dtype=jnp.float32)
    f1s, f2s = [], []
    for h in range(heads):
        hh = jnp.dot(x, w0_ref[h], preferred_element_type=jnp.float32)
        h0_ref[h] = jnp.concatenate([hh, ones], axis=1).astype(jnp.bfloat16)
        f1s.append(jnp.sum(hh * a1_ref[h][None, :], axis=1, keepdims=True))
        f2s.append(jnp.sum(hh * a2_ref[h][None, :], axis=1, keepdims=True))
    f1_ref[...] = jnp.concatenate(f1s, axis=1).astype(jnp.bfloat16)
    f2t_ref[...] = jnp.concatenate(f2s, axis=1).T.astype(jnp.bfloat16)


def _attn_block(mask, f1_col, f2_row, h_aug):
    z = f1_col + f2_row
    e = jnp.maximum(z, jnp.bfloat16(0.2) * z)
    e = jnp.where(mask, e, jnp.bfloat16(_NEG))
    m = jnp.max(e, axis=1, keepdims=True)
    p = jnp.exp(e - m)
    os = jnp.dot(p, h_aug, preferred_element_type=jnp.float32)
    f = h_aug.shape[1] - 1
    return os[:, :f] / os[:, f:]


def _layer1_kernel(adj_ref, h0_ref, f1_ref, f2t_ref, wc_ref, a1c_ref, a2c_ref,
                   hc_ref, f1c_ref, f2c_ref):
    mask = adj_ref[...] > 0
    heads = h0_ref.shape[0]
    outs = []
    for h in range(heads):
        outs.append(_attn_block(mask, f1_ref[:, h:h + 1], f2t_ref[h:h + 1, :],
                                h0_ref[h]))
    x = jnp.concatenate(outs, axis=1)
    x = jnp.where(x > 0, x, jnp.exp(x) - 1.0)
    hc = jnp.dot(x, wc_ref[...], preferred_element_type=jnp.float32)
    ones = jnp.ones((hc.shape[0], 1), dtype=jnp.float32)
    hc_ref[...] = jnp.concatenate([hc, ones], axis=1).astype(jnp.bfloat16)
    f1c = jnp.sum(hc * a1c_ref[...], axis=1, keepdims=True)
    f2c = jnp.sum(hc * a2c_ref[...], axis=1, keepdims=True)
    f1c_ref[...] = f1c.astype(jnp.bfloat16)
    f2c_ref[...] = f2c.astype(jnp.bfloat16)


def _cls_kernel(adj_ref, hc_ref, f1c_ref, f2c_ref, out_ref):
    mask = adj_ref[...] > 0
    f2c_row = f2c_ref[...].T
    out_ref[...] = _attn_block(mask, f1c_ref[...], f2c_row, hc_ref[...])


@jax.jit
def kernel(features, adj, W0, a1_0, a2_0, Wc, a1_c, a2_c):
    n, d_in = features.shape
    heads, _, f_out = W0.shape
    d_mid = heads * f_out
    n_cls = Wc.shape[1]

    h0, f1, f2t = pl.pallas_call(
        _proj_kernel,
        out_shape=[
            jax.ShapeDtypeStruct((heads, n, f_out + 1), jnp.bfloat16),
            jax.ShapeDtypeStruct((n, heads), jnp.bfloat16),
            jax.ShapeDtypeStruct((heads, n), jnp.bfloat16),
        ],
    )(features, W0, a1_0, a2_0)

    bi = _row_block(n, 80)
    hc, f1c, f2c = pl.pallas_call(
        _layer1_kernel,
        grid=(n // bi,),
        in_specs=[
            pl.BlockSpec((bi, n), lambda i: (i, 0)),
            pl.BlockSpec((heads, n, f_out + 1), lambda i: (0, 0, 0)),
            pl.BlockSpec((bi, heads), lambda i: (i, 0)),
            pl.BlockSpec((heads, n), lambda i: (0, 0)),
            pl.BlockSpec((d_mid, n_cls), lambda i: (0, 0)),
            pl.BlockSpec((1, n_cls), lambda i: (0, 0)),
            pl.BlockSpec((1, n_cls), lambda i: (0, 0)),
        ],
        out_specs=[
            pl.BlockSpec((bi, n_cls + 1), lambda i: (i, 0)),
            pl.BlockSpec((bi, 1), lambda i: (i, 0)),
            pl.BlockSpec((bi, 1), lambda i: (i, 0)),
        ],
        out_shape=[
            jax.ShapeDtypeStruct((n, n_cls + 1), jnp.bfloat16),
            jax.ShapeDtypeStruct((n, 1), jnp.bfloat16),
            jax.ShapeDtypeStruct((n, 1), jnp.bfloat16),
        ],
    )(adj, h0, f1, f2t, Wc, a1_c.reshape(1, n_cls), a2_c.reshape(1, n_cls))

    bc = _row_block(n, 200)
    out = pl.pallas_call(
        _cls_kernel,
        grid=(n // bc,),
        in_specs=[
            pl.BlockSpec((bc, n), lambda i: (i, 0)),
            pl.BlockSpec((n, n_cls + 1), lambda i: (0, 0)),
            pl.BlockSpec((bc, 1), lambda i: (i, 0)),
            pl.BlockSpec((n, 1), lambda i: (0, 0)),
        ],
        out_specs=pl.BlockSpec((bc, n_cls), lambda i: (i, 0)),
        out_shape=jax.ShapeDtypeStruct((n, n_cls), jnp.float32),
    )(adj, hc, f1c, f2c)
    return out


# speedup vs baseline: 4.6049x; 1.5331x over previous
import jax
import jax.numpy as jnp
from jax.experimental import pallas as pl

_NEG = -1e9
_LOG2E = 1.4426950408889634


def _row_block(n: int, target: int) -> int:
    best = 8
    for d in range(8, min(n, target) + 1, 8):
        if n % d == 0:
            best = d
    return best if n % best == 0 else n


def _proj_kernel(x_ref, w0_ref, a1_ref, a2_ref, h0_ref, f1_ref, f2t_ref):
    x = x_ref[...]
    heads = w0_ref.shape[0]
    n = x.shape[0]
    ones = jnp.ones((n, 1), dtype=jnp.float32)
    f1s, f2s = [], []
    for h in range(heads):
        hh = jnp.dot(x, w0_ref[h], preferred_element_type=jnp.float32)
        h0_ref[h] = jnp.concatenate([hh, ones], axis=1).astype(jnp.bfloat16)
        f1s.append(jnp.sum(hh * a1_ref[h][None, :], axis=1, keepdims=True))
        f2s.append(jnp.sum(hh * a2_ref[h][None, :], axis=1, keepdims=True))
    f1_ref[...] = (jnp.concatenate(f1s, axis=1) * _LOG2E).astype(jnp.bfloat16)
    f2t_ref[...] = (jnp.concatenate(f2s, axis=1).T * _LOG2E).astype(jnp.bfloat16)


def _attn_block(bias, f1_col, f2_row, h_aug):
    mhat = f1_col + jnp.max(f2_row, axis=1, keepdims=True)
    mhat = jnp.maximum(mhat, jnp.bfloat16(0.2) * mhat)
    z = f1_col + f2_row
    e = jnp.maximum(z, jnp.bfloat16(0.2) * z)
    p = jnp.exp2((e - mhat) + bias)
    os = jnp.dot(p, h_aug, preferred_element_type=jnp.float32)
    f = h_aug.shape[1] - 1
    return os[:, :f] / os[:, f:]


def _layer1_kernel(adj_ref, h0_ref, f1_ref, f2t_ref, wc_ref, a1c_ref, a2c_ref,
                   hc_ref, f1c_ref, f2c_ref):
    bias = jnp.where(adj_ref[...] > 0, 0.0, _NEG).astype(jnp.bfloat16)
    heads = h0_ref.shape[0]
    outs = []
    for h in range(heads):
        outs.append(_attn_block(bias, f1_ref[:, h:h + 1], f2t_ref[h:h + 1, :],
                                h0_ref[h]))
    x = jnp.concatenate(outs, axis=1)
    x = jnp.where(x > 0, x, jnp.exp(x) - 1.0)
    hc = jnp.dot(x, wc_ref[...], preferred_element_type=jnp.float32)
    ones = jnp.ones((hc.shape[0], 1), dtype=jnp.float32)
    hc_ref[...] = jnp.concatenate([hc, ones], axis=1).astype(jnp.bfloat16)
    f1c = jnp.sum(hc * a1c_ref[...], axis=1, keepdims=True)
    f2c = jnp.sum(hc * a2c_ref[...], axis=1, keepdims=True)
    f1c_ref[...] = (f1c * _LOG2E).astype(jnp.bfloat16)
    f2c_ref[...] = (f2c * _LOG2E).astype(jnp.bfloat16)


def _cls_kernel(adj_ref, hc_ref, f1c_ref, f2c_ref, out_ref):
    bias = jnp.where(adj_ref[...] > 0, 0.0, _NEG).astype(jnp.bfloat16)
    f2c_row = f2c_ref[...].T
    out_ref[...] = _attn_block(bias, f1c_ref[...], f2c_row, hc_ref[...])


@jax.jit
def kernel(features, adj, W0, a1_0, a2_0, Wc, a1_c, a2_c):
    n, d_in = features.shape
    heads, _, f_out = W0.shape
    d_mid = heads * f_out
    n_cls = Wc.shape[1]

    h0, f1, f2t = pl.pallas_call(
        _proj_kernel,
        out_shape=[
            jax.ShapeDtypeStruct((heads, n, f_out + 1), jnp.bfloat16),
            jax.ShapeDtypeStruct((n, heads), jnp.bfloat16),
            jax.ShapeDtypeStruct((heads, n), jnp.bfloat16),
        ],
    )(features, W0, a1_0, a2_0)

    bi = _row_block(n, 200)
    hc, f1c, f2c = pl.pallas_call(
        _layer1_kernel,
        grid=(n // bi,),
        in_specs=[
            pl.BlockSpec((bi, n), lambda i: (i, 0)),
            pl.BlockSpec((heads, n, f_out + 1), lambda i: (0, 0, 0)),
            pl.BlockSpec((bi, heads), lambda i: (i, 0)),
            pl.BlockSpec((heads, n), lambda i: (0, 0)),
            pl.BlockSpec((d_mid, n_cls), lambda i: (0, 0)),
            pl.BlockSpec((1, n_cls), lambda i: (0, 0)),
            pl.BlockSpec((1, n_cls), lambda i: (0, 0)),
        ],
        out_specs=[
            pl.BlockSpec((bi, n_cls + 1), lambda i: (i, 0)),
            pl.BlockSpec((bi, 1), lambda i: (i, 0)),
            pl.BlockSpec((bi, 1), lambda i: (i, 0)),
        ],
        out_shape=[
            jax.ShapeDtypeStruct((n, n_cls + 1), jnp.bfloat16),
            jax.ShapeDtypeStruct((n, 1), jnp.bfloat16),
            jax.ShapeDtypeStruct((n, 1), jnp.bfloat16),
        ],
    )(adj, h0, f1, f2t, Wc, a1_c.reshape(1, n_cls), a2_c.reshape(1, n_cls))

    bc = _row_block(n, 200)
    out = pl.pallas_call(
        _cls_kernel,
        grid=(n // bc,),
        in_specs=[
            pl.BlockSpec((bc, n), lambda i: (i, 0)),
            pl.BlockSpec((n, n_cls + 1), lambda i: (0, 0)),
            pl.BlockSpec((bc, 1), lambda i: (i, 0)),
            pl.BlockSpec((n, 1), lambda i: (0, 0)),
        ],
        out_specs=pl.BlockSpec((bc, n_cls), lambda i: (i, 0)),
        out_shape=jax.ShapeDtypeStruct((n, n_cls), jnp.float32),
    )(adj, hc, f1c, f2c)
    return out
